# Initial kernel scaffold; baseline (speedup 1.0000x reference)
import jax
import jax.numpy as jnp
from jax.experimental import pallas as pl


def kernel(nfeats, efeats, edge_index, W_e, b_e, W_n, b_n):
    raise NotImplementedError("write your pallas kernel here")


# speedup vs baseline: 3.8161x; 3.8161x over previous
import functools

import jax
import jax.numpy as jnp
from jax import lax
from jax.experimental import pallas as pl
from jax.experimental.pallas import tpu as pltpu
from jax.experimental.pallas import tpu_sc as plsc

N = 10000
E = 320000
D = 128

NC = 2
NS = 16
NW = NC * NS
EW = E // NW
CH = 80
NCHUNK = EW // CH
ZCH = 80
NZ = N // ZCH

_sc_mesh = plsc.VectorSubcoreMesh(core_axis_name="c", subcore_axis_name="s")


def _tables_body(nf_ref, w1_ref, w3_ref, be_ref, p_ref, q_ref):
    nf = nf_ref[...]
    p_ref[...] = jnp.dot(nf, w1_ref[...],
                         preferred_element_type=jnp.float32) + be_ref[...]
    q_ref[...] = jnp.dot(nf, w3_ref[...], preferred_element_type=jnp.float32)


def _tables(nf, w1, w3, be):
    return pl.pallas_call(
        _tables_body,
        out_shape=(jax.ShapeDtypeStruct((N, D), jnp.float32),
                   jax.ShapeDtypeStruct((N, D), jnp.float32)),
    )(nf, w1, w3, be)


@functools.partial(
    pl.kernel,
    out_type=(jax.ShapeDtypeStruct((E, D), jnp.float32),
              jax.ShapeDtypeStruct((E, D), jnp.float32)),
    mesh=_sc_mesh,
    scratch_types=[
        pltpu.VMEM((NCHUNK, CH), jnp.int32),
        pltpu.VMEM((NCHUNK, CH), jnp.int32),
        pltpu.VMEM((CH, D), jnp.float32),
        pltpu.VMEM((CH, D), jnp.float32),
        pltpu.SemaphoreType.DMA,
        pltpu.SemaphoreType.DMA,
    ],
)
def _gather(p_hbm, q_hbm, eidx_hbm, s1_hbm, s2_hbm,
            sidx_v, didx_v, rows1_v, rows2_v, sem1, sem2):
    c = lax.axis_index("c")
    s = lax.axis_index("s")
    wid = s * NC + c
    base = wid * EW

    pltpu.sync_copy(eidx_hbm.at[0, wid], sidx_v)
    pltpu.sync_copy(eidx_hbm.at[1, wid], didx_v)

    @pl.loop(0, NCHUNK)
    def _chunk(i):
        cp1 = pltpu.async_copy(p_hbm.at[sidx_v.at[i]], rows1_v, sem1)
        cp2 = pltpu.async_copy(q_hbm.at[didx_v.at[i]], rows2_v, sem2)
        cp1.wait()
        pltpu.sync_copy(rows1_v, s1_hbm.at[pl.ds(base + i * CH, CH)])
        cp2.wait()
        pltpu.sync_copy(rows2_v, s2_hbm.at[pl.ds(base + i * CH, CH)])


@functools.partial(
    pl.kernel,
    out_type=jax.ShapeDtypeStruct((NC, N, D), jnp.float32),
    mesh=_sc_mesh,
    scratch_types=[
        pltpu.VMEM((CH,), jnp.int32),
        pltpu.VMEM((CH, D), jnp.float32),
        pltpu.VMEM((ZCH, D), jnp.float32),
        pltpu.VMEM_SHARED((N, D), jnp.float32),
    ],
)
def _count(dst_hbm, cnts_hbm, cidx_v, ones_v, zrow_v, cnt_sh):
    c = lax.axis_index("c")
    s = lax.axis_index("s")
    wid = s * NC + c

    @pl.loop(0, CH)
    def _fill(i):
        @pl.loop(0, D // 16)
        def _fill_j(j):
            ones_v[i, pl.ds(j * 16, 16)] = jnp.full((16,), 1.0, jnp.float32)
            zrow_v[i, pl.ds(j * 16, 16)] = jnp.zeros((16,), jnp.float32)

    @pl.loop(0, (NZ + NS - 1) // NS)
    def _zero(t):
        k = t * NS + s

        @pl.when(k < NZ)
        def _():
            pltpu.sync_copy(zrow_v, cnt_sh.at[pl.ds(k * ZCH, ZCH)])

    plsc.subcore_barrier()

    @pl.loop(0, NCHUNK)
    def _chunk(i):
        pltpu.sync_copy(dst_hbm.at[wid, i], cidx_v)
        pltpu.sync_copy(ones_v, cnt_sh.at[cidx_v], add=True)

    plsc.subcore_barrier()

    @pl.loop(0, (NZ + NS - 1) // NS)
    def _writeback(t):
        k = t * NS + s

        @pl.when(k < NZ)
        def _():
            off = k * ZCH
            pltpu.sync_copy(cnt_sh.at[pl.ds(off, ZCH)],
                            cnts_hbm.at[c, pl.ds(off, ZCH)])


EB = 2000


def _edge_body(s1_ref, s2_ref, ef_ref, w2_ref, m_ref, mo_ref):
    ef = ef_ref[...]
    acc = s1_ref[...] + s2_ref[...] + jnp.dot(
        ef, w2_ref[...], preferred_element_type=jnp.float32)
    m = jnp.maximum(acc, 0.0)
    m_ref[...] = m
    mo_ref[...] = m + ef


def _edge(s1, s2, ef, w2):
    blk = pl.BlockSpec((EB, D), lambda i: (i, 0))
    wblk = pl.BlockSpec((D, D), lambda i: (0, 0))
    return pl.pallas_call(
        _edge_body,
        grid=(E // EB,),
        in_specs=[blk, blk, blk, wblk],
        out_specs=[blk, blk],
        out_shape=(jax.ShapeDtypeStruct((E, D), jnp.float32),
                   jax.ShapeDtypeStruct((E, D), jnp.float32)),
    )(s1, s2, ef, w2)


@functools.partial(
    pl.kernel,
    out_type=jax.ShapeDtypeStruct((NC, N, D), jnp.float32),
    mesh=_sc_mesh,
    scratch_types=[
        pltpu.VMEM((CH,), jnp.int32),
        pltpu.VMEM((CH, D), jnp.float32),
        pltpu.VMEM((ZCH, D), jnp.float32),
        pltpu.VMEM_SHARED((N, D), jnp.float32),
    ],
)
def _scatter(m_hbm, dst_hbm, sums_hbm, idx_v, rows_v, zrow_v, acc_sh):
    c = lax.axis_index("c")
    s = lax.axis_index("s")
    wid = s * NC + c

    @pl.loop(0, ZCH)
    def _fill_zrow(i):
        @pl.loop(0, D // 16)
        def _fill_zrow_j(j):
            zrow_v[i, pl.ds(j * 16, 16)] = jnp.zeros((16,), jnp.float32)

    @pl.loop(0, (NZ + NS - 1) // NS)
    def _zero(t):
        k = t * NS + s

        @pl.when(k < NZ)
        def _():
            pltpu.sync_copy(zrow_v, acc_sh.at[pl.ds(k * ZCH, ZCH)])

    plsc.subcore_barrier()

    @pl.loop(0, NCHUNK)
    def _chunk(i):
        pltpu.sync_copy(dst_hbm.at[wid, i], idx_v)
        pltpu.sync_copy(m_hbm.at[wid, i], rows_v)
        pltpu.sync_copy(rows_v, acc_sh.at[idx_v], add=True)

    plsc.subcore_barrier()

    @pl.loop(0, (NZ + NS - 1) // NS)
    def _writeback(t):
        k = t * NS + s

        @pl.when(k < NZ)
        def _():
            off = k * ZCH
            pltpu.sync_copy(acc_sh.at[pl.ds(off, ZCH)],
                            sums_hbm.at[c, pl.ds(off, ZCH)])


def _node_body(nf_ref, sums_ref, cnts_ref, wn1_ref, wn2_ref, bn_ref, out_ref):
    nf = nf_ref[...]
    sums = sums_ref[0] + sums_ref[1]
    cnt = cnts_ref[0, :, 0:1] + cnts_ref[1, :, 0:1]
    hagg = sums / jnp.maximum(cnt, 1.0)
    acc = (jnp.dot(nf, wn1_ref[...], preferred_element_type=jnp.float32)
           + jnp.dot(hagg, wn2_ref[...], preferred_element_type=jnp.float32)
           + bn_ref[...])
    out_ref[...] = jnp.maximum(acc, 0.0) + nf


def _node(nf, sums, cnts, wn1, wn2, bn):
    return pl.pallas_call(
        _node_body,
        out_shape=jax.ShapeDtypeStruct((N, D), jnp.float32),
    )(nf, sums, cnts, wn1, wn2, bn)


def kernel(nfeats, efeats, edge_index, W_e, b_e, W_n, b_n):
    w1 = W_e[:D]
    w2 = W_e[D:2 * D]
    w3 = W_e[2 * D:]
    wn1 = W_n[:D]
    wn2 = W_n[D:]
    be = b_e.reshape(1, D)
    bn = b_n.reshape(1, D)

    p, q = _tables(nfeats, w1, w3, be)
    eidx = edge_index.reshape(2, NW, NCHUNK, CH)
    s1, s2 = _gather(p, q, eidx)
    cnts = _count(eidx[1])
    m, m_out = _edge(s1, s2, efeats, w2)
    sums = _scatter(m.reshape(NW, NCHUNK, CH, D), eidx[1])
    h = _node(nfeats, sums, cnts, wn1, wn2, bn)
    return (h, m_out)


# speedup vs baseline: 3.8780x; 1.0162x over previous
import functools

import jax
import jax.numpy as jnp
from jax import lax
from jax.experimental import pallas as pl
from jax.experimental.pallas import tpu as pltpu
from jax.experimental.pallas import tpu_sc as plsc

N = 10000
E = 320000
D = 128

NC = 2
NS = 16
NW = NC * NS
EW = E // NW
CH = 40
NCHUNK = EW // CH
NPAIR = NCHUNK // 2
ZCH = 80
NZ = N // ZCH

_sc_mesh = plsc.VectorSubcoreMesh(core_axis_name="c", subcore_axis_name="s")


def _tables_body(nf_ref, w1_ref, w3_ref, be_ref, p_ref, q_ref):
    nf = nf_ref[...]
    p_ref[...] = jnp.dot(nf, w1_ref[...],
                         preferred_element_type=jnp.float32) + be_ref[...]
    q_ref[...] = jnp.dot(nf, w3_ref[...], preferred_element_type=jnp.float32)


def _tables(nf, w1, w3, be):
    return pl.pallas_call(
        _tables_body,
        out_shape=(jax.ShapeDtypeStruct((N, D), jnp.float32),
                   jax.ShapeDtypeStruct((N, D), jnp.float32)),
    )(nf, w1, w3, be)


@functools.partial(
    pl.kernel,
    out_type=(jax.ShapeDtypeStruct((E, D), jnp.float32),
              jax.ShapeDtypeStruct((E, D), jnp.float32)),
    mesh=_sc_mesh,
    scratch_types=[
        pltpu.VMEM((NCHUNK, CH), jnp.int32),
        pltpu.VMEM((NCHUNK, CH), jnp.int32),
        pltpu.VMEM((CH, D), jnp.float32),
        pltpu.VMEM((CH, D), jnp.float32),
        pltpu.VMEM((CH, D), jnp.float32),
        pltpu.VMEM((CH, D), jnp.float32),
        pltpu.SemaphoreType.DMA,
        pltpu.SemaphoreType.DMA,
        pltpu.SemaphoreType.DMA,
        pltpu.SemaphoreType.DMA,
        pltpu.SemaphoreType.DMA,
        pltpu.SemaphoreType.DMA,
        pltpu.SemaphoreType.DMA,
        pltpu.SemaphoreType.DMA,
    ],
)
def _gather(p_hbm, q_hbm, eidx_hbm, s1_hbm, s2_hbm,
            sidx_v, didx_v, pb0, pb1, qb0, qb1,
            gp0, gp1, gq0, gq1, w10, w11, w20, w21):
    c = lax.axis_index("c")
    s = lax.axis_index("s")
    wid = s * NC + c
    base = wid * EW

    pltpu.sync_copy(eidx_hbm.at[0, wid], sidx_v)
    pltpu.sync_copy(eidx_hbm.at[1, wid], didx_v)

    pltpu.async_copy(p_hbm.at[sidx_v.at[0]], pb0, gp0)
    pltpu.async_copy(q_hbm.at[didx_v.at[0]], qb0, gq0)

    @pl.loop(0, NPAIR)
    def _pair(t):
        i0 = t * 2
        i1 = i0 + 1
        pltpu.make_async_copy(p_hbm.at[sidx_v.at[i0]], pb0, gp0).wait()
        pltpu.make_async_copy(q_hbm.at[didx_v.at[i0]], qb0, gq0).wait()

        @pl.when(t > 0)
        def _drain_prev_writes():
            pltpu.make_async_copy(pb1, s1_hbm.at[pl.ds(base, CH)],
                                  w11).wait()
            pltpu.make_async_copy(qb1, s2_hbm.at[pl.ds(base, CH)],
                                  w21).wait()

        pltpu.async_copy(p_hbm.at[sidx_v.at[i1]], pb1, gp1)
        pltpu.async_copy(q_hbm.at[didx_v.at[i1]], qb1, gq1)
        cw10 = pltpu.async_copy(pb0, s1_hbm.at[pl.ds(base + i0 * CH, CH)],
                                w10)
        cw20 = pltpu.async_copy(qb0, s2_hbm.at[pl.ds(base + i0 * CH, CH)],
                                w20)

        pltpu.make_async_copy(p_hbm.at[sidx_v.at[i1]], pb1, gp1).wait()
        pltpu.make_async_copy(q_hbm.at[didx_v.at[i1]], qb1, gq1).wait()
        cw10.wait()
        cw20.wait()

        @pl.when(t + 1 < NPAIR)
        def _prefetch_next():
            i2 = i0 + 2
            pltpu.async_copy(p_hbm.at[sidx_v.at[i2]], pb0, gp0)
            pltpu.async_copy(q_hbm.at[didx_v.at[i2]], qb0, gq0)

        pltpu.async_copy(pb1, s1_hbm.at[pl.ds(base + i1 * CH, CH)], w11)
        pltpu.async_copy(qb1, s2_hbm.at[pl.ds(base + i1 * CH, CH)], w21)

    pltpu.make_async_copy(pb1, s1_hbm.at[pl.ds(base, CH)], w11).wait()
    pltpu.make_async_copy(qb1, s2_hbm.at[pl.ds(base, CH)], w21).wait()


@functools.partial(
    pl.kernel,
    out_type=jax.ShapeDtypeStruct((NC, N, D), jnp.float32),
    mesh=_sc_mesh,
    scratch_types=[
        pltpu.VMEM((CH,), jnp.int32),
        pltpu.VMEM((CH,), jnp.int32),
        pltpu.VMEM((CH, D), jnp.float32),
        pltpu.VMEM((ZCH, D), jnp.float32),
        pltpu.VMEM_SHARED((N, D), jnp.float32),
        pltpu.SemaphoreType.DMA,
        pltpu.SemaphoreType.DMA,
        pltpu.SemaphoreType.DMA,
        pltpu.SemaphoreType.DMA,
    ],
)
def _count(dst_hbm, cnts_hbm, cidx0, cidx1, ones_v, zrow_v, cnt_sh,
           xi0, xi1, sc0, sc1):
    c = lax.axis_index("c")
    s = lax.axis_index("s")
    wid = s * NC + c

    @pl.loop(0, CH)
    def _fill(i):
        @pl.loop(0, D // 16)
        def _fill_j(j):
            ones_v[i, pl.ds(j * 16, 16)] = jnp.full((16,), 1.0, jnp.float32)

    @pl.loop(0, ZCH)
    def _fill_z(i):
        @pl.loop(0, D // 16)
        def _fill_zj(j):
            zrow_v[i, pl.ds(j * 16, 16)] = jnp.zeros((16,), jnp.float32)

    @pl.loop(0, (NZ + NS - 1) // NS)
    def _zero(t):
        k = t * NS + s

        @pl.when(k < NZ)
        def _():
            pltpu.sync_copy(zrow_v, cnt_sh.at[pl.ds(k * ZCH, ZCH)])

    plsc.subcore_barrier()

    pltpu.async_copy(dst_hbm.at[wid, 0], cidx0, xi0)

    @pl.loop(0, NPAIR)
    def _pair(t):
        i0 = t * 2
        i1 = i0 + 1
        pltpu.make_async_copy(dst_hbm.at[wid, i0], cidx0, xi0).wait()

        @pl.when(t > 0)
        def _drain_prev():
            pltpu.make_async_copy(ones_v, cnt_sh.at[cidx1], sc1).wait()

        pltpu.async_copy(dst_hbm.at[wid, i1], cidx1, xi1)
        cs0 = pltpu.async_copy(ones_v, cnt_sh.at[cidx0], sc0, add=True)

        pltpu.make_async_copy(dst_hbm.at[wid, i1], cidx1, xi1).wait()
        cs0.wait()

        @pl.when(t + 1 < NPAIR)
        def _prefetch_next():
            pltpu.async_copy(dst_hbm.at[wid, i0 + 2], cidx0, xi0)

        pltpu.async_copy(ones_v, cnt_sh.at[cidx1], sc1, add=True)

    pltpu.make_async_copy(ones_v, cnt_sh.at[cidx1], sc1).wait()

    plsc.subcore_barrier()

    @pl.loop(0, (NZ + NS - 1) // NS)
    def _writeback(t):
        k = t * NS + s

        @pl.when(k < NZ)
        def _():
            off = k * ZCH
            pltpu.sync_copy(cnt_sh.at[pl.ds(off, ZCH)],
                            cnts_hbm.at[c, pl.ds(off, ZCH)])


EB = 2000


def _edge_body(s1_ref, s2_ref, ef_ref, w2_ref, m_ref, mo_ref):
    ef = ef_ref[...]
    acc = s1_ref[...] + s2_ref[...] + jnp.dot(
        ef, w2_ref[...], preferred_element_type=jnp.float32)
    m = jnp.maximum(acc, 0.0)
    m_ref[...] = m
    mo_ref[...] = m + ef


def _edge(s1, s2, ef, w2):
    blk = pl.BlockSpec((EB, D), lambda i: (i, 0))
    wblk = pl.BlockSpec((D, D), lambda i: (0, 0))
    return pl.pallas_call(
        _edge_body,
        grid=(E // EB,),
        in_specs=[blk, blk, blk, wblk],
        out_specs=[blk, blk],
        out_shape=(jax.ShapeDtypeStruct((E, D), jnp.float32),
                   jax.ShapeDtypeStruct((E, D), jnp.float32)),
    )(s1, s2, ef, w2)


@functools.partial(
    pl.kernel,
    out_type=jax.ShapeDtypeStruct((NC, N, D), jnp.float32),
    mesh=_sc_mesh,
    scratch_types=[
        pltpu.VMEM((CH,), jnp.int32),
        pltpu.VMEM((CH,), jnp.int32),
        pltpu.VMEM((CH, D), jnp.float32),
        pltpu.VMEM((CH, D), jnp.float32),
        pltpu.VMEM((ZCH, D), jnp.float32),
        pltpu.VMEM_SHARED((N, D), jnp.float32),
        pltpu.SemaphoreType.DMA,
        pltpu.SemaphoreType.DMA,
        pltpu.SemaphoreType.DMA,
        pltpu.SemaphoreType.DMA,
        pltpu.SemaphoreType.DMA,
        pltpu.SemaphoreType.DMA,
    ],
)
def _scatter(m_hbm, dst_hbm, sums_hbm,
             cidx0, cidx1, rb0, rb1, zrow_v, acc_sh,
             xi0, xi1, r0, r1, sc0, sc1):
    c = lax.axis_index("c")
    s = lax.axis_index("s")
    wid = s * NC + c

    @pl.loop(0, ZCH)
    def _fill_z(i):
        @pl.loop(0, D // 16)
        def _fill_zj(j):
            zrow_v[i, pl.ds(j * 16, 16)] = jnp.zeros((16,), jnp.float32)

    @pl.loop(0, (NZ + NS - 1) // NS)
    def _zero(t):
        k = t * NS + s

        @pl.when(k < NZ)
        def _():
            pltpu.sync_copy(zrow_v, acc_sh.at[pl.ds(k * ZCH, ZCH)])

    plsc.subcore_barrier()

    pltpu.async_copy(dst_hbm.at[wid, 0], cidx0, xi0)
    pltpu.async_copy(m_hbm.at[wid, 0], rb0, r0)

    @pl.loop(0, NPAIR)
    def _pair(t):
        i0 = t * 2
        i1 = i0 + 1
        pltpu.make_async_copy(dst_hbm.at[wid, i0], cidx0, xi0).wait()
        pltpu.make_async_copy(m_hbm.at[wid, i0], rb0, r0).wait()

        @pl.when(t > 0)
        def _drain_prev():
            pltpu.make_async_copy(rb1, acc_sh.at[cidx1], sc1).wait()

        pltpu.async_copy(dst_hbm.at[wid, i1], cidx1, xi1)
        pltpu.async_copy(m_hbm.at[wid, i1], rb1, r1)
        cs0 = pltpu.async_copy(rb0, acc_sh.at[cidx0], sc0, add=True)

        pltpu.make_async_copy(dst_hbm.at[wid, i1], cidx1, xi1).wait()
        pltpu.make_async_copy(m_hbm.at[wid, i1], rb1, r1).wait()
        cs0.wait()

        @pl.when(t + 1 < NPAIR)
        def _prefetch_next():
            i2 = i0 + 2
            pltpu.async_copy(dst_hbm.at[wid, i2], cidx0, xi0)
            pltpu.async_copy(m_hbm.at[wid, i2], rb0, r0)

        pltpu.async_copy(rb1, acc_sh.at[cidx1], sc1, add=True)

    pltpu.make_async_copy(rb1, acc_sh.at[cidx1], sc1).wait()

    plsc.subcore_barrier()

    @pl.loop(0, (NZ + NS - 1) // NS)
    def _writeback(t):
        k = t * NS + s

        @pl.when(k < NZ)
        def _():
            off = k * ZCH
            pltpu.sync_copy(acc_sh.at[pl.ds(off, ZCH)],
                            sums_hbm.at[c, pl.ds(off, ZCH)])


def _node_body(nf_ref, sums_ref, cnts_ref, wn1_ref, wn2_ref, bn_ref, out_ref):
    nf = nf_ref[...]
    sums = sums_ref[0] + sums_ref[1]
    cnt = cnts_ref[0, :, 0:1] + cnts_ref[1, :, 0:1]
    hagg = sums / jnp.maximum(cnt, 1.0)
    acc = (jnp.dot(nf, wn1_ref[...], preferred_element_type=jnp.float32)
           + jnp.dot(hagg, wn2_ref[...], preferred_element_type=jnp.float32)
           + bn_ref[...])
    out_ref[...] = jnp.maximum(acc, 0.0) + nf


def _node(nf, sums, cnts, wn1, wn2, bn):
    return pl.pallas_call(
        _node_body,
        out_shape=jax.ShapeDtypeStruct((N, D), jnp.float32),
    )(nf, sums, cnts, wn1, wn2, bn)


def kernel(nfeats, efeats, edge_index, W_e, b_e, W_n, b_n):
    w1 = W_e[:D]
    w2 = W_e[D:2 * D]
    w3 = W_e[2 * D:]
    wn1 = W_n[:D]
    wn2 = W_n[D:]
    be = b_e.reshape(1, D)
    bn = b_n.reshape(1, D)

    p, q = _tables(nfeats, w1, w3, be)
    eidx = edge_index.reshape(2, NW, NCHUNK, CH)
    s1, s2 = _gather(p, q, eidx)
    cnts = _count(eidx[1])
    m, m_out = _edge(s1, s2, efeats, w2)
    sums = _scatter(m.reshape(NW, NCHUNK, CH, D), eidx[1])
    h = _node(nfeats, sums, cnts, wn1, wn2, bn)
    return (h, m_out)


# speedup vs baseline: 3.8981x; 1.0052x over previous
import functools

import jax
import jax.numpy as jnp
from jax import lax
from jax.experimental import pallas as pl
from jax.experimental.pallas import tpu as pltpu
from jax.experimental.pallas import tpu_sc as plsc

N = 10000
E = 320000
D = 128

NC = 2
NS = 16
NW = NC * NS
EW = E // NW
CH = 40
NCHUNK = EW // CH
NPAIR = NCHUNK // 2
ZCH = 80
NZ = N // ZCH

_sc_mesh = plsc.VectorSubcoreMesh(core_axis_name="c", subcore_axis_name="s")


def _tables_body(nf_ref, w1_ref, w3_ref, be_ref, p_ref, q_ref):
    nf = nf_ref[...]
    p_ref[...] = jnp.dot(nf, w1_ref[...],
                         preferred_element_type=jnp.float32) + be_ref[...]
    q_ref[...] = jnp.dot(nf, w3_ref[...], preferred_element_type=jnp.float32)


def _tables(nf, w1, w3, be):
    return pl.pallas_call(
        _tables_body,
        out_shape=(jax.ShapeDtypeStruct((N, D), jnp.float32),
                   jax.ShapeDtypeStruct((N, D), jnp.float32)),
    )(nf, w1, w3, be)


@functools.partial(
    pl.kernel,
    out_type=(jax.ShapeDtypeStruct((E, D), jnp.float32),
              jax.ShapeDtypeStruct((E, D), jnp.float32)),
    mesh=_sc_mesh,
    scratch_types=[
        pltpu.VMEM((NCHUNK, CH), jnp.int32),
        pltpu.VMEM((NCHUNK, CH), jnp.int32),
        pltpu.VMEM((CH, D), jnp.float32),
        pltpu.VMEM((CH, D), jnp.float32),
        pltpu.VMEM((CH, D), jnp.float32),
        pltpu.VMEM((CH, D), jnp.float32),
        pltpu.SemaphoreType.DMA,
        pltpu.SemaphoreType.DMA,
        pltpu.SemaphoreType.DMA,
        pltpu.SemaphoreType.DMA,
        pltpu.SemaphoreType.DMA,
        pltpu.SemaphoreType.DMA,
        pltpu.SemaphoreType.DMA,
        pltpu.SemaphoreType.DMA,
    ],
)
def _gather(p_hbm, q_hbm, eidx_hbm, s1_hbm, s2_hbm,
            sidx_v, didx_v, pb0, pb1, qb0, qb1,
            gp0, gp1, gq0, gq1, w10, w11, w20, w21):
    c = lax.axis_index("c")
    s = lax.axis_index("s")
    wid = s * NC + c
    base = wid * EW

    pltpu.sync_copy(eidx_hbm.at[0, wid], sidx_v)
    pltpu.sync_copy(eidx_hbm.at[1, wid], didx_v)

    pltpu.async_copy(p_hbm.at[sidx_v.at[0]], pb0, gp0)
    pltpu.async_copy(q_hbm.at[didx_v.at[0]], qb0, gq0)

    @pl.loop(0, NPAIR)
    def _pair(t):
        i0 = t * 2
        i1 = i0 + 1
        pltpu.make_async_copy(p_hbm.at[sidx_v.at[i0]], pb0, gp0).wait()
        pltpu.make_async_copy(q_hbm.at[didx_v.at[i0]], qb0, gq0).wait()

        @pl.when(t > 0)
        def _drain_prev_writes():
            pltpu.make_async_copy(pb1, s1_hbm.at[pl.ds(base, CH)],
                                  w11).wait()
            pltpu.make_async_copy(qb1, s2_hbm.at[pl.ds(base, CH)],
                                  w21).wait()

        pltpu.async_copy(p_hbm.at[sidx_v.at[i1]], pb1, gp1)
        pltpu.async_copy(q_hbm.at[didx_v.at[i1]], qb1, gq1)
        cw10 = pltpu.async_copy(pb0, s1_hbm.at[pl.ds(base + i0 * CH, CH)],
                                w10)
        cw20 = pltpu.async_copy(qb0, s2_hbm.at[pl.ds(base + i0 * CH, CH)],
                                w20)

        pltpu.make_async_copy(p_hbm.at[sidx_v.at[i1]], pb1, gp1).wait()
        pltpu.make_async_copy(q_hbm.at[didx_v.at[i1]], qb1, gq1).wait()
        cw10.wait()
        cw20.wait()

        @pl.when(t + 1 < NPAIR)
        def _prefetch_next():
            i2 = i0 + 2
            pltpu.async_copy(p_hbm.at[sidx_v.at[i2]], pb0, gp0)
            pltpu.async_copy(q_hbm.at[didx_v.at[i2]], qb0, gq0)

        pltpu.async_copy(pb1, s1_hbm.at[pl.ds(base + i1 * CH, CH)], w11)
        pltpu.async_copy(qb1, s2_hbm.at[pl.ds(base + i1 * CH, CH)], w21)

    pltpu.make_async_copy(pb1, s1_hbm.at[pl.ds(base, CH)], w11).wait()
    pltpu.make_async_copy(qb1, s2_hbm.at[pl.ds(base, CH)], w21).wait()


@functools.partial(
    pl.kernel,
    out_type=jax.ShapeDtypeStruct((NW, N), jnp.float32),
    mesh=_sc_mesh,
    compiler_params=pltpu.CompilerParams(needs_layout_passes=False),
    scratch_types=[
        pltpu.VMEM((EW,), jnp.int32),
        pltpu.VMEM((N,), jnp.float32),
    ],
)
def _count(dst_hbm, cnts_hbm, idx_v, hist_v):
    c = lax.axis_index("c")
    s = lax.axis_index("s")
    wid = s * NC + c

    @pl.loop(0, N // 16)
    def _zero(i):
        hist_v[pl.ds(i * 16, 16)] = jnp.zeros((16,), jnp.float32)

    pltpu.sync_copy(dst_hbm.at[wid], idx_v)

    @pl.loop(0, EW // 16)
    def _step(j):
        v = idx_v[pl.ds(j * 16, 16)]
        cnt, last = plsc.scan_count(v)
        val = cnt.astype(jnp.float32)
        plsc.addupdate_scatter(hist_v, [v], val, mask=last)

    pltpu.sync_copy(hist_v, cnts_hbm.at[wid])


EB = 2000


def _edge_body(s1_ref, s2_ref, ef_ref, w2_ref, m_ref, mo_ref):
    ef = ef_ref[...]
    acc = s1_ref[...] + s2_ref[...] + jnp.dot(
        ef, w2_ref[...], preferred_element_type=jnp.float32)
    m = jnp.maximum(acc, 0.0)
    m_ref[...] = m
    mo_ref[...] = m + ef


def _edge(s1, s2, ef, w2):
    blk = pl.BlockSpec((EB, D), lambda i: (i, 0))
    wblk = pl.BlockSpec((D, D), lambda i: (0, 0))
    return pl.pallas_call(
        _edge_body,
        grid=(E // EB,),
        in_specs=[blk, blk, blk, wblk],
        out_specs=[blk, blk],
        out_shape=(jax.ShapeDtypeStruct((E, D), jnp.float32),
                   jax.ShapeDtypeStruct((E, D), jnp.float32)),
    )(s1, s2, ef, w2)


@functools.partial(
    pl.kernel,
    out_type=jax.ShapeDtypeStruct((NC, N, D), jnp.float32),
    mesh=_sc_mesh,
    scratch_types=[
        pltpu.VMEM((CH,), jnp.int32),
        pltpu.VMEM((CH,), jnp.int32),
        pltpu.VMEM((CH, D), jnp.float32),
        pltpu.VMEM((CH, D), jnp.float32),
        pltpu.VMEM((ZCH, D), jnp.float32),
        pltpu.VMEM_SHARED((N, D), jnp.float32),
        pltpu.SemaphoreType.DMA,
        pltpu.SemaphoreType.DMA,
        pltpu.SemaphoreType.DMA,
        pltpu.SemaphoreType.DMA,
        pltpu.SemaphoreType.DMA,
        pltpu.SemaphoreType.DMA,
    ],
)
def _scatter(m_hbm, dst_hbm, sums_hbm,
             cidx0, cidx1, rb0, rb1, zrow_v, acc_sh,
             xi0, xi1, r0, r1, sc0, sc1):
    c = lax.axis_index("c")
    s = lax.axis_index("s")
    wid = s * NC + c

    @pl.loop(0, ZCH)
    def _fill_z(i):
        @pl.loop(0, D // 16)
        def _fill_zj(j):
            zrow_v[i, pl.ds(j * 16, 16)] = jnp.zeros((16,), jnp.float32)

    @pl.loop(0, (NZ + NS - 1) // NS)
    def _zero(t):
        k = t * NS + s

        @pl.when(k < NZ)
        def _():
            pltpu.sync_copy(zrow_v, acc_sh.at[pl.ds(k * ZCH, ZCH)])

    plsc.subcore_barrier()

    pltpu.async_copy(dst_hbm.at[wid, 0], cidx0, xi0)
    pltpu.async_copy(m_hbm.at[wid, 0], rb0, r0)

    @pl.loop(0, NPAIR)
    def _pair(t):
        i0 = t * 2
        i1 = i0 + 1
        pltpu.make_async_copy(dst_hbm.at[wid, i0], cidx0, xi0).wait()
        pltpu.make_async_copy(m_hbm.at[wid, i0], rb0, r0).wait()

        @pl.when(t > 0)
        def _drain_prev():
            pltpu.make_async_copy(rb1, acc_sh.at[cidx1], sc1).wait()

        pltpu.async_copy(dst_hbm.at[wid, i1], cidx1, xi1)
        pltpu.async_copy(m_hbm.at[wid, i1], rb1, r1)
        cs0 = pltpu.async_copy(rb0, acc_sh.at[cidx0], sc0, add=True)

        pltpu.make_async_copy(dst_hbm.at[wid, i1], cidx1, xi1).wait()
        pltpu.make_async_copy(m_hbm.at[wid, i1], rb1, r1).wait()
        cs0.wait()

        @pl.when(t + 1 < NPAIR)
        def _prefetch_next():
            i2 = i0 + 2
            pltpu.async_copy(dst_hbm.at[wid, i2], cidx0, xi0)
            pltpu.async_copy(m_hbm.at[wid, i2], rb0, r0)

        pltpu.async_copy(rb1, acc_sh.at[cidx1], sc1, add=True)

    pltpu.make_async_copy(rb1, acc_sh.at[cidx1], sc1).wait()

    plsc.subcore_barrier()

    @pl.loop(0, (NZ + NS - 1) // NS)
    def _writeback(t):
        k = t * NS + s

        @pl.when(k < NZ)
        def _():
            off = k * ZCH
            pltpu.sync_copy(acc_sh.at[pl.ds(off, ZCH)],
                            sums_hbm.at[c, pl.ds(off, ZCH)])


def _node_body(nf_ref, sums_ref, cnts_ref, wn1_ref, wn2_ref, bn_ref, out_ref):
    nf = nf_ref[...]
    sums = sums_ref[0] + sums_ref[1]
    cnt = jnp.sum(cnts_ref[...], axis=0).reshape(N, 1)
    hagg = sums / jnp.maximum(cnt, 1.0)
    acc = (jnp.dot(nf, wn1_ref[...], preferred_element_type=jnp.float32)
           + jnp.dot(hagg, wn2_ref[...], preferred_element_type=jnp.float32)
           + bn_ref[...])
    out_ref[...] = jnp.maximum(acc, 0.0) + nf


def _node(nf, sums, cnts, wn1, wn2, bn):
    return pl.pallas_call(
        _node_body,
        out_shape=jax.ShapeDtypeStruct((N, D), jnp.float32),
    )(nf, sums, cnts, wn1, wn2, bn)


def kernel(nfeats, efeats, edge_index, W_e, b_e, W_n, b_n):
    w1 = W_e[:D]
    w2 = W_e[D:2 * D]
    w3 = W_e[2 * D:]
    wn1 = W_n[:D]
    wn2 = W_n[D:]
    be = b_e.reshape(1, D)
    bn = b_n.reshape(1, D)

    p, q = _tables(nfeats, w1, w3, be)
    eidx = edge_index.reshape(2, NW, NCHUNK, CH)
    s1, s2 = _gather(p, q, eidx)
    cnts = _count(edge_index[1].reshape(NW, EW))
    m, m_out = _edge(s1, s2, efeats, w2)
    sums = _scatter(m.reshape(NW, NCHUNK, CH, D), eidx[1])
    h = _node(nfeats, sums, cnts, wn1, wn2, bn)
    return (h, m_out)


# speedup vs baseline: 4.2174x; 1.0819x over previous
import functools

import jax
import jax.numpy as jnp
from jax import lax
from jax.experimental import pallas as pl
from jax.experimental.pallas import tpu as pltpu
from jax.experimental.pallas import tpu_sc as plsc

N = 10000
E = 320000
D = 128

NC = 2
NS = 16
NW = NC * NS
EW = E // NW
CH = 40
NCHUNK = EW // CH
NPAIR = NCHUNK // 2
ZCH = 80
NZ = N // ZCH

_sc_mesh = plsc.VectorSubcoreMesh(core_axis_name="c", subcore_axis_name="s")


def _tables_body(nf_ref, w1_ref, w3_ref, be_ref, p_ref, q_ref):
    nf = nf_ref[...]
    p_ref[...] = jnp.dot(nf, w1_ref[...],
                         preferred_element_type=jnp.float32) + be_ref[...]
    q_ref[...] = jnp.dot(nf, w3_ref[...], preferred_element_type=jnp.float32)


def _tables(nf, w1, w3, be):
    return pl.pallas_call(
        _tables_body,
        out_shape=(jax.ShapeDtypeStruct((N, D), jnp.float32),
                   jax.ShapeDtypeStruct((N, D), jnp.float32)),
    )(nf, w1, w3, be)


@functools.partial(
    pl.kernel,
    out_type=jax.ShapeDtypeStruct((E, D), jnp.float32),
    mesh=_sc_mesh,
    scratch_types=[
        pltpu.VMEM((NCHUNK, CH), jnp.int32),
        pltpu.VMEM((NCHUNK, CH), jnp.int32),
        pltpu.VMEM((CH, D), jnp.float32),
        pltpu.VMEM((CH, D), jnp.float32),
        pltpu.VMEM((CH, D), jnp.float32),
        pltpu.VMEM((CH, D), jnp.float32),
        pltpu.VMEM((CH, D), jnp.float32),
        pltpu.VMEM((CH, D), jnp.float32),
        pltpu.SemaphoreType.DMA,
        pltpu.SemaphoreType.DMA,
        pltpu.SemaphoreType.DMA,
        pltpu.SemaphoreType.DMA,
        pltpu.SemaphoreType.DMA,
        pltpu.SemaphoreType.DMA,
    ],
)
def _gather(p_hbm, q_hbm, eidx_hbm, s_hbm,
            sidx_v, didx_v, pb0, pb1, qb0, qb1, sb0, sb1,
            gp0, gp1, gq0, gq1, w0, w1):
    c = lax.axis_index("c")
    s = lax.axis_index("s")
    wid = s * NC + c
    base = wid * EW

    pltpu.sync_copy(eidx_hbm.at[0, wid], sidx_v)
    pltpu.sync_copy(eidx_hbm.at[1, wid], didx_v)

    pltpu.async_copy(p_hbm.at[sidx_v.at[0]], pb0, gp0)
    pltpu.async_copy(q_hbm.at[didx_v.at[0]], qb0, gq0)

    def _add_rows(dst, a, b):
        @pl.loop(0, CH)
        def _row(i):
            for j in range(D // 16):
                sl = pl.ds(j * 16, 16)
                dst[i, sl] = a[i, sl] + b[i, sl]

    @pl.loop(0, NPAIR)
    def _pair(t):
        i0 = t * 2
        i1 = i0 + 1
        pltpu.make_async_copy(p_hbm.at[sidx_v.at[i0]], pb0, gp0).wait()
        pltpu.make_async_copy(q_hbm.at[didx_v.at[i0]], qb0, gq0).wait()
        pltpu.async_copy(p_hbm.at[sidx_v.at[i1]], pb1, gp1)
        pltpu.async_copy(q_hbm.at[didx_v.at[i1]], qb1, gq1)

        @pl.when(t > 0)
        def _drain_w0():
            pltpu.make_async_copy(sb0, s_hbm.at[pl.ds(base, CH)], w0).wait()

        _add_rows(sb0, pb0, qb0)
        pltpu.async_copy(sb0, s_hbm.at[pl.ds(base + i0 * CH, CH)], w0)

        pltpu.make_async_copy(p_hbm.at[sidx_v.at[i1]], pb1, gp1).wait()
        pltpu.make_async_copy(q_hbm.at[didx_v.at[i1]], qb1, gq1).wait()

        @pl.when(t + 1 < NPAIR)
        def _prefetch_next():
            i2 = i0 + 2
            pltpu.async_copy(p_hbm.at[sidx_v.at[i2]], pb0, gp0)
            pltpu.async_copy(q_hbm.at[didx_v.at[i2]], qb0, gq0)

        @pl.when(t > 0)
        def _drain_w1():
            pltpu.make_async_copy(sb1, s_hbm.at[pl.ds(base, CH)], w1).wait()

        _add_rows(sb1, pb1, qb1)
        pltpu.async_copy(sb1, s_hbm.at[pl.ds(base + i1 * CH, CH)], w1)

    pltpu.make_async_copy(sb0, s_hbm.at[pl.ds(base, CH)], w0).wait()
    pltpu.make_async_copy(sb1, s_hbm.at[pl.ds(base, CH)], w1).wait()


@functools.partial(
    pl.kernel,
    out_type=jax.ShapeDtypeStruct((NW, N), jnp.float32),
    mesh=_sc_mesh,
    compiler_params=pltpu.CompilerParams(needs_layout_passes=False),
    scratch_types=[
        pltpu.VMEM((EW,), jnp.int32),
        pltpu.VMEM((N,), jnp.float32),
    ],
)
def _count(dst_hbm, cnts_hbm, idx_v, hist_v):
    c = lax.axis_index("c")
    s = lax.axis_index("s")
    wid = s * NC + c

    @pl.loop(0, N // 16)
    def _zero(i):
        hist_v[pl.ds(i * 16, 16)] = jnp.zeros((16,), jnp.float32)

    pltpu.sync_copy(dst_hbm.at[wid], idx_v)

    @pl.loop(0, EW // 16)
    def _step(j):
        v = idx_v[pl.ds(j * 16, 16)]
        cnt, last = plsc.scan_count(v)
        val = cnt.astype(jnp.float32)
        plsc.addupdate_scatter(hist_v, [v], val, mask=last)

    pltpu.sync_copy(hist_v, cnts_hbm.at[wid])


EB = 2000


def _edge_body(s_ref, ef_ref, w2_ref, m_ref, mo_ref):
    ef = ef_ref[...]
    acc = s_ref[...] + jnp.dot(
        ef, w2_ref[...], preferred_element_type=jnp.float32)
    m = jnp.maximum(acc, 0.0)
    m_ref[...] = m
    mo_ref[...] = m + ef


def _edge(sv, ef, w2):
    blk = pl.BlockSpec((EB, D), lambda i: (i, 0))
    wblk = pl.BlockSpec((D, D), lambda i: (0, 0))
    return pl.pallas_call(
        _edge_body,
        grid=(E // EB,),
        in_specs=[blk, blk, wblk],
        out_specs=[blk, blk],
        out_shape=(jax.ShapeDtypeStruct((E, D), jnp.float32),
                   jax.ShapeDtypeStruct((E, D), jnp.float32)),
    )(sv, ef, w2)


@functools.partial(
    pl.kernel,
    out_type=jax.ShapeDtypeStruct((NC, N, D), jnp.float32),
    mesh=_sc_mesh,
    scratch_types=[
        pltpu.VMEM((CH,), jnp.int32),
        pltpu.VMEM((CH,), jnp.int32),
        pltpu.VMEM((CH, D), jnp.float32),
        pltpu.VMEM((CH, D), jnp.float32),
        pltpu.VMEM((ZCH, D), jnp.float32),
        pltpu.VMEM_SHARED((N, D), jnp.float32),
        pltpu.SemaphoreType.DMA,
        pltpu.SemaphoreType.DMA,
        pltpu.SemaphoreType.DMA,
        pltpu.SemaphoreType.DMA,
        pltpu.SemaphoreType.DMA,
        pltpu.SemaphoreType.DMA,
    ],
)
def _scatter(m_hbm, dst_hbm, sums_hbm,
             cidx0, cidx1, rb0, rb1, zrow_v, acc_sh,
             xi0, xi1, r0, r1, sc0, sc1):
    c = lax.axis_index("c")
    s = lax.axis_index("s")
    wid = s * NC + c

    @pl.loop(0, ZCH)
    def _fill_z(i):
        @pl.loop(0, D // 16)
        def _fill_zj(j):
            zrow_v[i, pl.ds(j * 16, 16)] = jnp.zeros((16,), jnp.float32)

    @pl.loop(0, (NZ + NS - 1) // NS)
    def _zero(t):
        k = t * NS + s

        @pl.when(k < NZ)
        def _():
            pltpu.sync_copy(zrow_v, acc_sh.at[pl.ds(k * ZCH, ZCH)])

    plsc.subcore_barrier()

    pltpu.async_copy(dst_hbm.at[wid, 0], cidx0, xi0)
    pltpu.async_copy(m_hbm.at[wid, 0], rb0, r0)

    @pl.loop(0, NPAIR)
    def _pair(t):
        i0 = t * 2
        i1 = i0 + 1
        pltpu.make_async_copy(dst_hbm.at[wid, i0], cidx0, xi0).wait()
        pltpu.make_async_copy(m_hbm.at[wid, i0], rb0, r0).wait()

        @pl.when(t > 0)
        def _drain_prev():
            pltpu.make_async_copy(rb1, acc_sh.at[cidx1], sc1).wait()

        pltpu.async_copy(dst_hbm.at[wid, i1], cidx1, xi1)
        pltpu.async_copy(m_hbm.at[wid, i1], rb1, r1)
        cs0 = pltpu.async_copy(rb0, acc_sh.at[cidx0], sc0, add=True)

        pltpu.make_async_copy(dst_hbm.at[wid, i1], cidx1, xi1).wait()
        pltpu.make_async_copy(m_hbm.at[wid, i1], rb1, r1).wait()
        cs0.wait()

        @pl.when(t + 1 < NPAIR)
        def _prefetch_next():
            i2 = i0 + 2
            pltpu.async_copy(dst_hbm.at[wid, i2], cidx0, xi0)
            pltpu.async_copy(m_hbm.at[wid, i2], rb0, r0)

        pltpu.async_copy(rb1, acc_sh.at[cidx1], sc1, add=True)

    pltpu.make_async_copy(rb1, acc_sh.at[cidx1], sc1).wait()

    plsc.subcore_barrier()

    @pl.loop(0, (NZ + NS - 1) // NS)
    def _writeback(t):
        k = t * NS + s

        @pl.when(k < NZ)
        def _():
            off = k * ZCH
            pltpu.sync_copy(acc_sh.at[pl.ds(off, ZCH)],
                            sums_hbm.at[c, pl.ds(off, ZCH)])


def _node_body(nf_ref, sums_ref, cnts_ref, wn1_ref, wn2_ref, bn_ref, out_ref):
    nf = nf_ref[...]
    sums = sums_ref[0] + sums_ref[1]
    cnt = jnp.sum(cnts_ref[...], axis=0).reshape(N, 1)
    hagg = sums / jnp.maximum(cnt, 1.0)
    acc = (jnp.dot(nf, wn1_ref[...], preferred_element_type=jnp.float32)
           + jnp.dot(hagg, wn2_ref[...], preferred_element_type=jnp.float32)
           + bn_ref[...])
    out_ref[...] = jnp.maximum(acc, 0.0) + nf


def _node(nf, sums, cnts, wn1, wn2, bn):
    return pl.pallas_call(
        _node_body,
        out_shape=jax.ShapeDtypeStruct((N, D), jnp.float32),
    )(nf, sums, cnts, wn1, wn2, bn)


def kernel(nfeats, efeats, edge_index, W_e, b_e, W_n, b_n):
    w1 = W_e[:D]
    w2 = W_e[D:2 * D]
    w3 = W_e[2 * D:]
    wn1 = W_n[:D]
    wn2 = W_n[D:]
    be = b_e.reshape(1, D)
    bn = b_n.reshape(1, D)

    p, q = _tables(nfeats, w1, w3, be)
    eidx = edge_index.reshape(2, NW, NCHUNK, CH)
    sv = _gather(p, q, eidx)
    cnts = _count(edge_index[1].reshape(NW, EW))
    m, m_out = _edge(sv, efeats, w2)
    sums = _scatter(m.reshape(NW, NCHUNK, CH, D), eidx[1])
    h = _node(nfeats, sums, cnts, wn1, wn2, bn)
    return (h, m_out)


# speedup vs baseline: 4.3664x; 1.0353x over previous
import functools

import jax
import jax.numpy as jnp
from jax import lax
from jax.experimental import pallas as pl
from jax.experimental.pallas import tpu as pltpu
from jax.experimental.pallas import tpu_sc as plsc

N = 10000
E = 320000
D = 128

NC = 2
NS = 16
NW = NC * NS
EW = E // NW
CH = 40
NCHUNK = EW // CH
NPAIR = NCHUNK // 2
ZCH = 80
NZ = N // ZCH

_sc_mesh = plsc.VectorSubcoreMesh(core_axis_name="c", subcore_axis_name="s")


def _tables_body(nf_ref, w1_ref, w3_ref, be_ref, p_ref, q_ref):
    nf = nf_ref[...]
    p_ref[...] = jnp.dot(nf, w1_ref[...],
                         preferred_element_type=jnp.float32) + be_ref[...]
    q_ref[...] = jnp.dot(nf, w3_ref[...], preferred_element_type=jnp.float32)


def _tables(nf, w1, w3, be):
    return pl.pallas_call(
        _tables_body,
        out_shape=(jax.ShapeDtypeStruct((N, D), jnp.float32),
                   jax.ShapeDtypeStruct((N, D), jnp.float32)),
    )(nf, w1, w3, be)


@functools.partial(
    pl.kernel,
    out_type=jax.ShapeDtypeStruct((E, D), jnp.float32),
    mesh=_sc_mesh,
    scratch_types=[
        pltpu.VMEM((NCHUNK, CH), jnp.int32),
        pltpu.VMEM((NCHUNK, CH), jnp.int32),
        pltpu.VMEM((CH, D), jnp.float32),
        pltpu.VMEM((CH, D), jnp.float32),
        pltpu.VMEM((CH, D), jnp.float32),
        pltpu.VMEM((CH, D), jnp.float32),
        pltpu.VMEM((CH, D), jnp.float32),
        pltpu.VMEM((CH, D), jnp.float32),
        pltpu.SemaphoreType.DMA,
        pltpu.SemaphoreType.DMA,
        pltpu.SemaphoreType.DMA,
        pltpu.SemaphoreType.DMA,
        pltpu.SemaphoreType.DMA,
        pltpu.SemaphoreType.DMA,
    ],
)
def _gather(p_hbm, q_hbm, eidx_hbm, s_hbm,
            sidx_v, didx_v, pb0, pb1, qb0, qb1, sb0, sb1,
            gp0, gp1, gq0, gq1, w0, w1):
    c = lax.axis_index("c")
    s = lax.axis_index("s")
    wid = s * NC + c
    base = wid * EW

    pltpu.sync_copy(eidx_hbm.at[0, wid], sidx_v)
    pltpu.sync_copy(eidx_hbm.at[1, wid], didx_v)

    pltpu.async_copy(p_hbm.at[sidx_v.at[0]], pb0, gp0)
    pltpu.async_copy(q_hbm.at[didx_v.at[0]], qb0, gq0)

    def _add_rows(dst, a, b):
        @pl.loop(0, CH)
        def _row(i):
            for j in range(D // 16):
                sl = pl.ds(j * 16, 16)
                dst[i, sl] = a[i, sl] + b[i, sl]

    @pl.loop(0, NPAIR)
    def _pair(t):
        i0 = t * 2
        i1 = i0 + 1
        pltpu.make_async_copy(p_hbm.at[sidx_v.at[i0]], pb0, gp0).wait()
        pltpu.make_async_copy(q_hbm.at[didx_v.at[i0]], qb0, gq0).wait()
        pltpu.async_copy(p_hbm.at[sidx_v.at[i1]], pb1, gp1)
        pltpu.async_copy(q_hbm.at[didx_v.at[i1]], qb1, gq1)

        @pl.when(t > 0)
        def _drain_w0():
            pltpu.make_async_copy(sb0, s_hbm.at[pl.ds(base, CH)], w0).wait()

        _add_rows(sb0, pb0, qb0)
        pltpu.async_copy(sb0, s_hbm.at[pl.ds(base + i0 * CH, CH)], w0)

        pltpu.make_async_copy(p_hbm.at[sidx_v.at[i1]], pb1, gp1).wait()
        pltpu.make_async_copy(q_hbm.at[didx_v.at[i1]], qb1, gq1).wait()

        @pl.when(t + 1 < NPAIR)
        def _prefetch_next():
            i2 = i0 + 2
            pltpu.async_copy(p_hbm.at[sidx_v.at[i2]], pb0, gp0)
            pltpu.async_copy(q_hbm.at[didx_v.at[i2]], qb0, gq0)

        @pl.when(t > 0)
        def _drain_w1():
            pltpu.make_async_copy(sb1, s_hbm.at[pl.ds(base, CH)], w1).wait()

        _add_rows(sb1, pb1, qb1)
        pltpu.async_copy(sb1, s_hbm.at[pl.ds(base + i1 * CH, CH)], w1)

    pltpu.make_async_copy(sb0, s_hbm.at[pl.ds(base, CH)], w0).wait()
    pltpu.make_async_copy(sb1, s_hbm.at[pl.ds(base, CH)], w1).wait()


@functools.partial(
    pl.kernel,
    out_type=jax.ShapeDtypeStruct((NW, N), jnp.float32),
    mesh=_sc_mesh,
    compiler_params=pltpu.CompilerParams(needs_layout_passes=False),
    scratch_types=[
        pltpu.VMEM((EW,), jnp.int32),
        pltpu.VMEM((N,), jnp.float32),
    ],
)
def _count(dst_hbm, cnts_hbm, idx_v, hist_v):
    c = lax.axis_index("c")
    s = lax.axis_index("s")
    wid = s * NC + c

    @pl.loop(0, N // 16)
    def _zero(i):
        hist_v[pl.ds(i * 16, 16)] = jnp.zeros((16,), jnp.float32)

    pltpu.sync_copy(dst_hbm.at[wid], idx_v)

    @pl.loop(0, EW // 16)
    def _step(j):
        v = idx_v[pl.ds(j * 16, 16)]
        cnt, last = plsc.scan_count(v)
        val = cnt.astype(jnp.float32)
        plsc.addupdate_scatter(hist_v, [v], val, mask=last)

    pltpu.sync_copy(hist_v, cnts_hbm.at[wid])


EB = 4000


def _edge_body(s_ref, ef_ref, w2_ref, m_ref, mo_ref):
    ef = ef_ref[...]
    acc = s_ref[...] + jnp.dot(
        ef, w2_ref[...], preferred_element_type=jnp.float32)
    m = jnp.maximum(acc, 0.0)
    m_ref[...] = m
    mo_ref[...] = m + ef


def _edge(sv, ef, w2):
    blk = pl.BlockSpec((EB, D), lambda i: (i, 0))
    wblk = pl.BlockSpec((D, D), lambda i: (0, 0))
    return pl.pallas_call(
        _edge_body,
        grid=(E // EB,),
        in_specs=[blk, blk, wblk],
        out_specs=[blk, blk],
        out_shape=(jax.ShapeDtypeStruct((E, D), jnp.float32),
                   jax.ShapeDtypeStruct((E, D), jnp.float32)),
    )(sv, ef, w2)


@functools.partial(
    pl.kernel,
    out_type=jax.ShapeDtypeStruct((NC, N, D), jnp.float32),
    mesh=_sc_mesh,
    scratch_types=[
        pltpu.VMEM((CH,), jnp.int32),
        pltpu.VMEM((CH,), jnp.int32),
        pltpu.VMEM((CH, D), jnp.float32),
        pltpu.VMEM((CH, D), jnp.float32),
        pltpu.VMEM((ZCH, D), jnp.float32),
        pltpu.VMEM_SHARED((N, D), jnp.float32),
        pltpu.SemaphoreType.DMA,
        pltpu.SemaphoreType.DMA,
        pltpu.SemaphoreType.DMA,
        pltpu.SemaphoreType.DMA,
        pltpu.SemaphoreType.DMA,
        pltpu.SemaphoreType.DMA,
    ],
)
def _scatter(m_hbm, dst_hbm, sums_hbm,
             cidx0, cidx1, rb0, rb1, zrow_v, acc_sh,
             xi0, xi1, r0, r1, sc0, sc1):
    c = lax.axis_index("c")
    s = lax.axis_index("s")
    wid = s * NC + c

    @pl.loop(0, ZCH)
    def _fill_z(i):
        @pl.loop(0, D // 16)
        def _fill_zj(j):
            zrow_v[i, pl.ds(j * 16, 16)] = jnp.zeros((16,), jnp.float32)

    @pl.loop(0, (NZ + NS - 1) // NS)
    def _zero(t):
        k = t * NS + s

        @pl.when(k < NZ)
        def _():
            pltpu.sync_copy(zrow_v, acc_sh.at[pl.ds(k * ZCH, ZCH)])

    plsc.subcore_barrier()

    pltpu.async_copy(dst_hbm.at[wid, 0], cidx0, xi0)
    pltpu.async_copy(m_hbm.at[wid, 0], rb0, r0)

    @pl.loop(0, NPAIR)
    def _pair(t):
        i0 = t * 2
        i1 = i0 + 1
        pltpu.make_async_copy(dst_hbm.at[wid, i0], cidx0, xi0).wait()
        pltpu.make_async_copy(m_hbm.at[wid, i0], rb0, r0).wait()

        @pl.when(t > 0)
        def _drain_prev():
            pltpu.make_async_copy(rb1, acc_sh.at[cidx1], sc1).wait()

        pltpu.async_copy(dst_hbm.at[wid, i1], cidx1, xi1)
        pltpu.async_copy(m_hbm.at[wid, i1], rb1, r1)
        cs0 = pltpu.async_copy(rb0, acc_sh.at[cidx0], sc0, add=True)

        pltpu.make_async_copy(dst_hbm.at[wid, i1], cidx1, xi1).wait()
        pltpu.make_async_copy(m_hbm.at[wid, i1], rb1, r1).wait()
        cs0.wait()

        @pl.when(t + 1 < NPAIR)
        def _prefetch_next():
            i2 = i0 + 2
            pltpu.async_copy(dst_hbm.at[wid, i2], cidx0, xi0)
            pltpu.async_copy(m_hbm.at[wid, i2], rb0, r0)

        pltpu.async_copy(rb1, acc_sh.at[cidx1], sc1, add=True)

    pltpu.make_async_copy(rb1, acc_sh.at[cidx1], sc1).wait()

    plsc.subcore_barrier()

    @pl.loop(0, (NZ + NS - 1) // NS)
    def _writeback(t):
        k = t * NS + s

        @pl.when(k < NZ)
        def _():
            off = k * ZCH
            pltpu.sync_copy(acc_sh.at[pl.ds(off, ZCH)],
                            sums_hbm.at[c, pl.ds(off, ZCH)])


def _node_body(nf_ref, sums_ref, cnts_ref, wn1_ref, wn2_ref, bn_ref, out_ref):
    nf = nf_ref[...]
    sums = sums_ref[0] + sums_ref[1]
    cnt = jnp.sum(cnts_ref[...], axis=0).reshape(N, 1)
    hagg = sums / jnp.maximum(cnt, 1.0)
    acc = (jnp.dot(nf, wn1_ref[...], preferred_element_type=jnp.float32)
           + jnp.dot(hagg, wn2_ref[...], preferred_element_type=jnp.float32)
           + bn_ref[...])
    out_ref[...] = jnp.maximum(acc, 0.0) + nf


def _node(nf, sums, cnts, wn1, wn2, bn):
    return pl.pallas_call(
        _node_body,
        out_shape=jax.ShapeDtypeStruct((N, D), jnp.float32),
    )(nf, sums, cnts, wn1, wn2, bn)


def kernel(nfeats, efeats, edge_index, W_e, b_e, W_n, b_n):
    w1 = W_e[:D]
    w2 = W_e[D:2 * D]
    w3 = W_e[2 * D:]
    wn1 = W_n[:D]
    wn2 = W_n[D:]
    be = b_e.reshape(1, D)
    bn = b_n.reshape(1, D)

    p, q = _tables(nfeats, w1, w3, be)
    eidx = edge_index.reshape(2, NW, NCHUNK, CH)
    sv = _gather(p, q, eidx)
    cnts = _count(edge_index[1].reshape(NW, EW))
    m, m_out = _edge(sv, efeats, w2)
    sums = _scatter(m.reshape(NW, NCHUNK, CH, D), eidx[1])
    h = _node(nfeats, sums, cnts, wn1, wn2, bn)
    return (h, m_out)


# speedup vs baseline: 4.3914x; 1.0057x over previous
import functools

import jax
import jax.numpy as jnp
from jax import lax
from jax.experimental import pallas as pl
from jax.experimental.pallas import tpu as pltpu
from jax.experimental.pallas import tpu_sc as plsc

N = 10000
E = 320000
D = 128

NC = 2
NS = 16
NW = NC * NS
EW = E // NW
CH = 40
NCHUNK = EW // CH
NPAIR = NCHUNK // 2
ZCH = 80
NZ = N // ZCH

_sc_mesh = plsc.VectorSubcoreMesh(core_axis_name="c", subcore_axis_name="s")


def _tables_body(nf_ref, w1_ref, w3_ref, be_ref, p_ref, q_ref):
    nf = nf_ref[...]
    p_ref[...] = jnp.dot(nf, w1_ref[...],
                         preferred_element_type=jnp.float32) + be_ref[...]
    q_ref[...] = jnp.dot(nf, w3_ref[...], preferred_element_type=jnp.float32)


def _tables(nf, w1, w3, be):
    return pl.pallas_call(
        _tables_body,
        out_shape=(jax.ShapeDtypeStruct((N, D), jnp.float32),
                   jax.ShapeDtypeStruct((N, D), jnp.float32)),
    )(nf, w1, w3, be)


@functools.partial(
    pl.kernel,
    out_type=jax.ShapeDtypeStruct((E, D), jnp.float32),
    mesh=_sc_mesh,
    scratch_types=[
        pltpu.VMEM((NCHUNK, CH), jnp.int32),
        pltpu.VMEM((NCHUNK, CH), jnp.int32),
        pltpu.VMEM((CH, D), jnp.float32),
        pltpu.VMEM((CH, D), jnp.float32),
        pltpu.VMEM((CH, D), jnp.float32),
        pltpu.VMEM((CH, D), jnp.float32),
        pltpu.VMEM((CH, D), jnp.float32),
        pltpu.VMEM((CH, D), jnp.float32),
        pltpu.SemaphoreType.DMA,
        pltpu.SemaphoreType.DMA,
        pltpu.SemaphoreType.DMA,
        pltpu.SemaphoreType.DMA,
        pltpu.SemaphoreType.DMA,
        pltpu.SemaphoreType.DMA,
    ],
)
def _gather(p_hbm, q_hbm, eidx_hbm, s_hbm,
            sidx_v, didx_v, pb0, pb1, qb0, qb1, sb0, sb1,
            gp0, gp1, gq0, gq1, w0, w1):
    c = lax.axis_index("c")
    s = lax.axis_index("s")
    wid = s * NC + c
    base = wid * EW

    pltpu.sync_copy(eidx_hbm.at[0, wid], sidx_v)
    pltpu.sync_copy(eidx_hbm.at[1, wid], didx_v)

    pltpu.async_copy(p_hbm.at[sidx_v.at[0]], pb0, gp0)
    pltpu.async_copy(q_hbm.at[didx_v.at[0]], qb0, gq0)

    def _add_rows(dst, a, b):
        @pl.loop(0, CH)
        def _row(i):
            for j in range(D // 16):
                sl = pl.ds(j * 16, 16)
                dst[i, sl] = a[i, sl] + b[i, sl]

    @pl.loop(0, NPAIR)
    def _pair(t):
        i0 = t * 2
        i1 = i0 + 1
        pltpu.make_async_copy(p_hbm.at[sidx_v.at[i0]], pb0, gp0).wait()
        pltpu.make_async_copy(q_hbm.at[didx_v.at[i0]], qb0, gq0).wait()
        pltpu.async_copy(p_hbm.at[sidx_v.at[i1]], pb1, gp1)
        pltpu.async_copy(q_hbm.at[didx_v.at[i1]], qb1, gq1)

        @pl.when(t > 0)
        def _drain_w0():
            pltpu.make_async_copy(sb0, s_hbm.at[pl.ds(base, CH)], w0).wait()

        _add_rows(sb0, pb0, qb0)
        pltpu.async_copy(sb0, s_hbm.at[pl.ds(base + i0 * CH, CH)], w0)

        pltpu.make_async_copy(p_hbm.at[sidx_v.at[i1]], pb1, gp1).wait()
        pltpu.make_async_copy(q_hbm.at[didx_v.at[i1]], qb1, gq1).wait()

        @pl.when(t + 1 < NPAIR)
        def _prefetch_next():
            i2 = i0 + 2
            pltpu.async_copy(p_hbm.at[sidx_v.at[i2]], pb0, gp0)
            pltpu.async_copy(q_hbm.at[didx_v.at[i2]], qb0, gq0)

        @pl.when(t > 0)
        def _drain_w1():
            pltpu.make_async_copy(sb1, s_hbm.at[pl.ds(base, CH)], w1).wait()

        _add_rows(sb1, pb1, qb1)
        pltpu.async_copy(sb1, s_hbm.at[pl.ds(base + i1 * CH, CH)], w1)

    pltpu.make_async_copy(sb0, s_hbm.at[pl.ds(base, CH)], w0).wait()
    pltpu.make_async_copy(sb1, s_hbm.at[pl.ds(base, CH)], w1).wait()


@functools.partial(
    pl.kernel,
    out_type=jax.ShapeDtypeStruct((NW, N), jnp.float32),
    mesh=_sc_mesh,
    compiler_params=pltpu.CompilerParams(needs_layout_passes=False),
    scratch_types=[
        pltpu.VMEM((EW,), jnp.int32),
        pltpu.VMEM((N,), jnp.float32),
    ],
)
def _count(dst_hbm, cnts_hbm, idx_v, hist_v):
    c = lax.axis_index("c")
    s = lax.axis_index("s")
    wid = s * NC + c

    @pl.loop(0, N // 16)
    def _zero(i):
        hist_v[pl.ds(i * 16, 16)] = jnp.zeros((16,), jnp.float32)

    pltpu.sync_copy(dst_hbm.at[wid], idx_v)

    @pl.loop(0, EW // 16)
    def _step(j):
        v = idx_v[pl.ds(j * 16, 16)]
        cnt, last = plsc.scan_count(v)
        val = cnt.astype(jnp.float32)
        plsc.addupdate_scatter(hist_v, [v], val, mask=last)

    pltpu.sync_copy(hist_v, cnts_hbm.at[wid])


EB = 8000


def _edge_body(s_ref, ef_ref, w2_ref, m_ref, mo_ref):
    ef = ef_ref[...]
    acc = s_ref[...] + jnp.dot(
        ef, w2_ref[...], preferred_element_type=jnp.float32)
    m = jnp.maximum(acc, 0.0)
    m_ref[...] = m
    mo_ref[...] = m + ef


def _edge(sv, ef, w2):
    blk = pl.BlockSpec((EB, D), lambda i: (i, 0))
    wblk = pl.BlockSpec((D, D), lambda i: (0, 0))
    return pl.pallas_call(
        _edge_body,
        grid=(E // EB,),
        in_specs=[blk, blk, wblk],
        out_specs=[blk, blk],
        out_shape=(jax.ShapeDtypeStruct((E, D), jnp.float32),
                   jax.ShapeDtypeStruct((E, D), jnp.float32)),
    )(sv, ef, w2)


@functools.partial(
    pl.kernel,
    out_type=jax.ShapeDtypeStruct((NC, N, D), jnp.float32),
    mesh=_sc_mesh,
    scratch_types=[
        pltpu.VMEM((CH,), jnp.int32),
        pltpu.VMEM((CH,), jnp.int32),
        pltpu.VMEM((CH, D), jnp.float32),
        pltpu.VMEM((CH, D), jnp.float32),
        pltpu.VMEM((ZCH, D), jnp.float32),
        pltpu.VMEM_SHARED((N, D), jnp.float32),
        pltpu.SemaphoreType.DMA,
        pltpu.SemaphoreType.DMA,
        pltpu.SemaphoreType.DMA,
        pltpu.SemaphoreType.DMA,
        pltpu.SemaphoreType.DMA,
        pltpu.SemaphoreType.DMA,
    ],
)
def _scatter(m_hbm, dst_hbm, sums_hbm,
             cidx0, cidx1, rb0, rb1, zrow_v, acc_sh,
             xi0, xi1, r0, r1, sc0, sc1):
    c = lax.axis_index("c")
    s = lax.axis_index("s")
    wid = s * NC + c

    @pl.loop(0, ZCH)
    def _fill_z(i):
        @pl.loop(0, D // 16)
        def _fill_zj(j):
            zrow_v[i, pl.ds(j * 16, 16)] = jnp.zeros((16,), jnp.float32)

    @pl.loop(0, (NZ + NS - 1) // NS)
    def _zero(t):
        k = t * NS + s

        @pl.when(k < NZ)
        def _():
            pltpu.sync_copy(zrow_v, acc_sh.at[pl.ds(k * ZCH, ZCH)])

    plsc.subcore_barrier()

    pltpu.async_copy(dst_hbm.at[wid, 0], cidx0, xi0)
    pltpu.async_copy(m_hbm.at[wid, 0], rb0, r0)

    @pl.loop(0, NPAIR)
    def _pair(t):
        i0 = t * 2
        i1 = i0 + 1
        pltpu.make_async_copy(dst_hbm.at[wid, i0], cidx0, xi0).wait()
        pltpu.make_async_copy(m_hbm.at[wid, i0], rb0, r0).wait()

        @pl.when(t > 0)
        def _drain_prev():
            pltpu.make_async_copy(rb1, acc_sh.at[cidx1], sc1).wait()

        pltpu.async_copy(dst_hbm.at[wid, i1], cidx1, xi1)
        pltpu.async_copy(m_hbm.at[wid, i1], rb1, r1)
        cs0 = pltpu.async_copy(rb0, acc_sh.at[cidx0], sc0, add=True)

        pltpu.make_async_copy(dst_hbm.at[wid, i1], cidx1, xi1).wait()
        pltpu.make_async_copy(m_hbm.at[wid, i1], rb1, r1).wait()
        cs0.wait()

        @pl.when(t + 1 < NPAIR)
        def _prefetch_next():
            i2 = i0 + 2
            pltpu.async_copy(dst_hbm.at[wid, i2], cidx0, xi0)
            pltpu.async_copy(m_hbm.at[wid, i2], rb0, r0)

        pltpu.async_copy(rb1, acc_sh.at[cidx1], sc1, add=True)

    pltpu.make_async_copy(rb1, acc_sh.at[cidx1], sc1).wait()

    plsc.subcore_barrier()

    @pl.loop(0, (NZ + NS - 1) // NS)
    def _writeback(t):
        k = t * NS + s

        @pl.when(k < NZ)
        def _():
            off = k * ZCH
            pltpu.sync_copy(acc_sh.at[pl.ds(off, ZCH)],
                            sums_hbm.at[c, pl.ds(off, ZCH)])


def _node_body(nf_ref, sums_ref, cnts_ref, wn1_ref, wn2_ref, bn_ref, out_ref):
    nf = nf_ref[...]
    sums = sums_ref[0] + sums_ref[1]
    cnt = jnp.sum(cnts_ref[...], axis=0).reshape(N, 1)
    hagg = sums / jnp.maximum(cnt, 1.0)
    acc = (jnp.dot(nf, wn1_ref[...], preferred_element_type=jnp.float32)
           + jnp.dot(hagg, wn2_ref[...], preferred_element_type=jnp.float32)
           + bn_ref[...])
    out_ref[...] = jnp.maximum(acc, 0.0) + nf


def _node(nf, sums, cnts, wn1, wn2, bn):
    return pl.pallas_call(
        _node_body,
        out_shape=jax.ShapeDtypeStruct((N, D), jnp.float32),
    )(nf, sums, cnts, wn1, wn2, bn)


def kernel(nfeats, efeats, edge_index, W_e, b_e, W_n, b_n):
    w1 = W_e[:D]
    w2 = W_e[D:2 * D]
    w3 = W_e[2 * D:]
    wn1 = W_n[:D]
    wn2 = W_n[D:]
    be = b_e.reshape(1, D)
    bn = b_n.reshape(1, D)

    p, q = _tables(nfeats, w1, w3, be)
    eidx = edge_index.reshape(2, NW, NCHUNK, CH)
    sv = _gather(p, q, eidx)
    cnts = _count(edge_index[1].reshape(NW, EW))
    m, m_out = _edge(sv, efeats, w2)
    sums = _scatter(m.reshape(NW, NCHUNK, CH, D), eidx[1])
    h = _node(nfeats, sums, cnts, wn1, wn2, bn)
    return (h, m_out)
